# baseline (speedup 1.0000x reference)
import functools

import jax
import jax.numpy as jnp
from jax import lax
from jax.experimental import pallas as pl
from jax.experimental.pallas import tpu as pltpu
from jax.experimental.pallas import tpu_sc as plsc

_B_TOTAL = 16384 * 26
_D = 16

_info = plsc.get_sparse_core_info()
_NC, _NS = _info.num_cores, _info.num_subcores
_NW = _NC * _NS
_B_PER_W = _B_TOTAL // _NW
_CHUNK = 1664
_NCHUNK = _B_PER_W // _CHUNK

_mesh = plsc.VectorSubcoreMesh(core_axis_name="c", subcore_axis_name="s")


@functools.partial(
    pl.kernel,
    mesh=_mesh,
    out_type=jax.ShapeDtypeStruct((_B_TOTAL, _D), jnp.float32),
    scratch_types=[
        pltpu.VMEM((_CHUNK,), jnp.int32),
        pltpu.VMEM((_CHUNK, _D), jnp.float32),
        pltpu.SemaphoreType.DMA,
    ],
    compiler_params=pltpu.CompilerParams(use_tc_tiling_on_sc=False),
)
def _gather_rows(idx_hbm, table_hbm, out_hbm, idx_v, rows_v, sem):
    wid = lax.axis_index("s") * _NC + lax.axis_index("c")
    base = wid * _B_PER_W

    def body(i, _):
        off = base + i * _CHUNK
        pltpu.sync_copy(idx_hbm.at[pl.ds(off, _CHUNK)], idx_v)
        pltpu.async_copy(table_hbm.at[idx_v], rows_v, sem).wait()
        pltpu.sync_copy(rows_v, out_hbm.at[pl.ds(off, _CHUNK)])
        return 0

    lax.fori_loop(0, _NCHUNK, body, 0)


def kernel(x, table):
    idx = x.reshape(-1).astype(jnp.int32)
    out = _gather_rows(idx, table)
    return out.reshape(x.shape[0], x.shape[1], _D)


# speedup vs baseline: 1.0105x; 1.0105x over previous
import functools

import jax
import jax.numpy as jnp
from jax import lax
from jax.experimental import pallas as pl
from jax.experimental.pallas import tpu as pltpu
from jax.experimental.pallas import tpu_sc as plsc

_B_TOTAL = 16384 * 26
_D = 16

_info = plsc.get_sparse_core_info()
_NC, _NS = _info.num_cores, _info.num_subcores
_NW = _NC * _NS
_B_PER_W = _B_TOTAL // _NW
_CHUNK = 1664
_NCHUNK = _B_PER_W // _CHUNK
_NBUF = 4

_mesh = plsc.VectorSubcoreMesh(core_axis_name="c", subcore_axis_name="s")


@functools.partial(
    pl.kernel,
    mesh=_mesh,
    out_type=jax.ShapeDtypeStruct((_B_TOTAL, _D), jnp.float32),
    scratch_types=[
        pltpu.VMEM((_B_PER_W,), jnp.int32),
        pltpu.VMEM((_NBUF, _CHUNK, _D), jnp.float32),
        pltpu.SemaphoreType.DMA((_NBUF,)),
        pltpu.SemaphoreType.DMA((_NBUF,)),
    ],
    compiler_params=pltpu.CompilerParams(use_tc_tiling_on_sc=False),
)
def _gather_rows(idx_hbm, table_hbm, out_hbm, idx_v, rows_v, gsem, ssem):
    wid = lax.axis_index("s") * _NC + lax.axis_index("c")
    base = wid * _B_PER_W
    pltpu.sync_copy(idx_hbm.at[pl.ds(base, _B_PER_W)], idx_v)

    gathers = [None] * _NCHUNK
    stores = [None] * _NCHUNK

    def start_gather(i):
        b = i % _NBUF
        if i >= _NBUF:
            stores[i - _NBUF].wait()
        gathers[i] = pltpu.async_copy(
            table_hbm.at[idx_v.at[pl.ds(i * _CHUNK, _CHUNK)]],
            rows_v.at[b],
            gsem.at[b],
        )

    for i in range(min(_NBUF, _NCHUNK)):
        start_gather(i)
    for i in range(_NCHUNK):
        gathers[i].wait()
        stores[i] = pltpu.async_copy(
            rows_v.at[i % _NBUF],
            out_hbm.at[pl.ds(base + i * _CHUNK, _CHUNK)],
            ssem.at[i % _NBUF],
        )
        if i + _NBUF < _NCHUNK:
            start_gather(i + _NBUF)
    for i in range(max(0, _NCHUNK - _NBUF), _NCHUNK):
        stores[i].wait()


def kernel(x, table):
    idx = x.reshape(-1).astype(jnp.int32)
    out = _gather_rows(idx, table)
    return out.reshape(x.shape[0], x.shape[1], _D)


# speedup vs baseline: 1.5952x; 1.5786x over previous
import functools

import jax
import jax.numpy as jnp
from jax import lax
from jax.experimental import pallas as pl
from jax.experimental.pallas import tpu as pltpu
from jax.experimental.pallas import tpu_sc as plsc

_BATCH = 16384
_NF = 26
_D = 16

_info = plsc.get_sparse_core_info()
_NC, _NS = _info.num_cores, _info.num_subcores
_NW = _NC * _NS
_TILES_PER_W = _BATCH // (128 * _NW)
_CHUNK_B = 64
_CHUNK = _CHUNK_B * _NF
_NCHUNK = 2 * _TILES_PER_W
_B_PER_W = _NCHUNK * _CHUNK

_mesh = plsc.VectorSubcoreMesh(core_axis_name="c", subcore_axis_name="s")


@functools.partial(
    pl.kernel,
    mesh=_mesh,
    out_type=jax.ShapeDtypeStruct((_NF, 2, _BATCH // 128, 8, 128), jnp.float32),
    scratch_types=[
        pltpu.VMEM((_B_PER_W,), jnp.int32),
        pltpu.VMEM((2, _CHUNK, _D), jnp.float32),
        pltpu.VMEM((_NF, 2, 8, _CHUNK_B), jnp.float32),
        pltpu.SemaphoreType.DMA((2,)),
        pltpu.SemaphoreType.DMA,
    ],
    compiler_params=pltpu.CompilerParams(
        use_tc_tiling_on_sc=False, needs_layout_passes=False
    ),
)
def _embed(idx_hbm, table_hbm, out_hbm, idx_v, rows_v, t_v, gsem, ssem):
    wid = lax.axis_index("s") * _NC + lax.axis_index("c")
    base = wid * _B_PER_W
    pltpu.sync_copy(idx_hbm.at[pl.ds(base, _B_PER_W)], idx_v)

    lane = lax.iota(jnp.int32, 16)
    dt_i = lax.shift_right_logical(lane, 3)
    dr_i = lax.bitwise_and(lane, 7)

    def start_gather(h):
        return pltpu.async_copy(
            table_hbm.at[idx_v.at[pl.ds(h * _CHUNK, _CHUNK)]],
            rows_v.at[h % 2],
            gsem.at[h % 2],
        )

    def transpose_chunk(rows_b):
        def f_body(f, _):
            f_vec = jnp.broadcast_to(f, (16,))

            def jg_body(jg, _):
                n0 = f + 104 * jg
                for u in range(4):
                    v = rows_b[n0 + 26 * u, :]
                    j_vec = jnp.broadcast_to(jg * 4 + u, (16,))
                    plsc.store_scatter(t_v, [f_vec, dt_i, dr_i, j_vec], v)
                return 0

            lax.fori_loop(0, _CHUNK_B // 4, jg_body, 0)
            return 0

        lax.fori_loop(0, _NF, f_body, 0)

    gathers = [None, None]
    gathers[0] = start_gather(0)
    prev_stores = []
    for h in range(_NCHUNK):
        gathers[h % 2].wait()
        if h + 1 < _NCHUNK:
            gathers[(h + 1) % 2] = start_gather(h + 1)
        for s in prev_stores:
            s.wait()
        transpose_chunk(rows_v.at[h % 2])
        bc = wid * _TILES_PER_W + h // 2
        bl0 = (h % 2) * _CHUNK_B
        prev_stores = [
            pltpu.async_copy(
                t_v.at[f],
                out_hbm.at[f, :, bc, :, pl.ds(bl0, _CHUNK_B)],
                ssem,
            )
            for f in range(_NF)
        ]
    for s in prev_stores:
        s.wait()


def kernel(x, table):
    idx = x.reshape(-1).astype(jnp.int32)
    o = _embed(idx, table)
    return o.transpose(2, 4, 0, 1, 3).reshape(_BATCH, _NF, _D)
